# Initial kernel scaffold; baseline (speedup 1.0000x reference)
import functools

import jax
import jax.numpy as jnp
from jax import lax
from jax.experimental import pallas as pl
from jax.experimental.pallas import tpu as pltpu
from jax.experimental.pallas import tpu_sc as plsc

N_NODES = 10000
D = 128
N_PAD = 10240
ROWS_PER_SUBCORE = N_PAD // 16
CHUNK = 128
NW = 32
N_EDGES = 320000
CHUNKS_PER_W = (N_EDGES + NW * CHUNK - 1) // (NW * CHUNK)
EPW = CHUNKS_PER_W * CHUNK
E_PAD = EPW * NW

_MESH = plsc.VectorSubcoreMesh(core_axis_name="c", subcore_axis_name="s")


@functools.partial(
    pl.kernel,
    out_type=jax.ShapeDtypeStruct((2, N_PAD), jnp.float32),
    mesh=_MESH,
    scratch_types=[
        pltpu.VMEM((CHUNK,), jnp.int32),
        pltpu.VMEM((CHUNK,), jnp.float32),
        pltpu.VMEM_SHARED((N_PAD,), jnp.float32),
    ],
)
def _sc_degree(dst_hbm, ones_hbm, zeros_hbm, out_hbm, dstv, onesv, hist_sh):
    cid = lax.axis_index("c")
    sid = lax.axis_index("s")
    wid = cid * 16 + sid
    row0 = sid * ROWS_PER_SUBCORE
    pltpu.sync_copy(ones_hbm, onesv)
    pltpu.sync_copy(zeros_hbm, hist_sh.at[pl.ds(row0, ROWS_PER_SUBCORE)])
    plsc.subcore_barrier()

    def body(j, carry):
        base = wid * EPW + j * CHUNK
        pltpu.sync_copy(dst_hbm.at[pl.ds(base, CHUNK)], dstv)
        pltpu.sync_copy(onesv, hist_sh.at[dstv], add=True)
        return carry

    lax.fori_loop(0, CHUNKS_PER_W, body, 0)
    plsc.subcore_barrier()
    pltpu.sync_copy(hist_sh.at[pl.ds(row0, ROWS_PER_SUBCORE)],
                    out_hbm.at[cid, pl.ds(row0, ROWS_PER_SUBCORE)])


@functools.partial(
    pl.kernel,
    out_type=jax.ShapeDtypeStruct((2, N_PAD, D), jnp.float32),
    mesh=_MESH,
    scratch_types=[
        pltpu.VMEM((CHUNK,), jnp.int32),
        pltpu.VMEM((CHUNK,), jnp.int32),
        pltpu.VMEM((CHUNK, D), jnp.float32),
        pltpu.VMEM_SHARED((N_PAD, D), jnp.float32),
        pltpu.SemaphoreType.DMA,
    ],
)
def _sc_aggregate(y_hbm, src_hbm, dst_hbm, zeros_hbm, out_hbm,
                  srcv, dstv, rows, acc_sh, sem):
    cid = lax.axis_index("c")
    sid = lax.axis_index("s")
    wid = cid * 16 + sid
    row0 = sid * ROWS_PER_SUBCORE
    pltpu.sync_copy(zeros_hbm, acc_sh.at[pl.ds(row0, ROWS_PER_SUBCORE)])
    plsc.subcore_barrier()

    def body(j, carry):
        base = wid * EPW + j * CHUNK
        pltpu.sync_copy(src_hbm.at[pl.ds(base, CHUNK)], srcv)
        pltpu.sync_copy(dst_hbm.at[pl.ds(base, CHUNK)], dstv)
        pltpu.async_copy(y_hbm.at[srcv], rows, sem).wait()
        pltpu.sync_copy(rows, acc_sh.at[dstv], add=True)
        return carry

    lax.fori_loop(0, CHUNKS_PER_W, body, 0)
    plsc.subcore_barrier()
    pltpu.sync_copy(acc_sh.at[pl.ds(row0, ROWS_PER_SUBCORE)],
                    out_hbm.at[cid, pl.ds(row0, ROWS_PER_SUBCORE)])


_BLK = 1024
_GRID = N_PAD // _BLK


def _dinv_of(hist_blk):
    deg = hist_blk[0, :] + hist_blk[1, :] + 1.0
    return lax.rsqrt(deg)


def _tc_first_body(hist_ref, x_ref, w_ref, y_ref):
    dinv = _dinv_of(hist_ref[...])
    h = jnp.dot(x_ref[...], w_ref[...], preferred_element_type=jnp.float32)
    y_ref[...] = h * dinv[:, None]


def _tc_mid_body(hist_ref, agg_ref, y_ref, b_ref, w_ref, o_ref):
    dinv = _dinv_of(hist_ref[...])
    s = agg_ref[0] + agg_ref[1] + y_ref[...]
    x1 = jnp.maximum(s * dinv[:, None] + b_ref[...], 0.0)
    o_ref[...] = jnp.dot(x1, w_ref[...], preferred_element_type=jnp.float32) * dinv[:, None]


def _tc_last_body(hist_ref, agg_ref, y_ref, b_ref, o_ref):
    dinv = _dinv_of(hist_ref[...])
    s = agg_ref[0] + agg_ref[1] + y_ref[...]
    o_ref[...] = s * dinv[:, None] + b_ref[...]


_hist_spec = pl.BlockSpec((2, _BLK), lambda i: (0, i))
_row_spec = pl.BlockSpec((_BLK, D), lambda i: (i, 0))
_agg_spec = pl.BlockSpec((2, _BLK, D), lambda i: (0, i, 0))
_full_spec = pl.BlockSpec((D, D), lambda i: (0, 0))
_bias_spec = pl.BlockSpec((1, D), lambda i: (0, 0))
_out_shape = jax.ShapeDtypeStruct((N_PAD, D), jnp.float32)

_tc_first = pl.pallas_call(
    _tc_first_body, grid=(_GRID,),
    in_specs=[_hist_spec, _row_spec, _full_spec],
    out_specs=_row_spec, out_shape=_out_shape)

_tc_mid = pl.pallas_call(
    _tc_mid_body, grid=(_GRID,),
    in_specs=[_hist_spec, _agg_spec, _row_spec, _bias_spec, _full_spec],
    out_specs=_row_spec, out_shape=_out_shape)

_tc_last = pl.pallas_call(
    _tc_last_body, grid=(_GRID,),
    in_specs=[_hist_spec, _agg_spec, _row_spec, _bias_spec],
    out_specs=_row_spec, out_shape=_out_shape)


def kernel(emb, edge_index, W1, b1, W2, b2):
    src = edge_index[0].astype(jnp.int32)
    dst = edge_index[1].astype(jnp.int32)

    n_extra = E_PAD - N_EDGES
    pad_src = (jnp.arange(n_extra, dtype=jnp.int32) * 97) % N_NODES
    pad_dst = N_NODES + (jnp.arange(n_extra, dtype=jnp.int32) % (N_PAD - N_NODES))
    src_p = jnp.concatenate([src, pad_src])
    dst_p = jnp.concatenate([dst, pad_dst])

    emb_p = jnp.pad(emb, ((0, N_PAD - N_NODES), (0, 0)))
    ones128 = jnp.ones((CHUNK,), jnp.float32)
    zrow = jnp.zeros((ROWS_PER_SUBCORE,), jnp.float32)
    zblk = jnp.zeros((ROWS_PER_SUBCORE, D), jnp.float32)
    b1r = b1.reshape(1, D)
    b2r = b2.reshape(1, D)

    hist = _sc_degree(dst_p, ones128, zrow)
    y1 = _tc_first(hist, emb_p, W1)
    agg1 = _sc_aggregate(y1, src_p, dst_p, zblk)
    y2 = _tc_mid(hist, agg1, y1, b1r, W2)
    agg2 = _sc_aggregate(y2, src_p, dst_p, zblk)
    out = _tc_last(hist, agg2, y2, b2r)
    return out[:N_NODES]


# speedup vs baseline: 16.2025x; 16.2025x over previous
import functools

import jax
import jax.numpy as jnp
from jax import lax
from jax.experimental import pallas as pl
from jax.experimental.pallas import tpu as pltpu
from jax.experimental.pallas import tpu_sc as plsc

N_NODES = 10000
D = 128
N_PAD = 10240
ROWS_PER_SUBCORE = N_PAD // 16
CHUNK = 128
NW = 32
N_EDGES = 320000
CHUNKS_PER_W = (N_EDGES + NW * CHUNK - 1) // (NW * CHUNK)
EPW = CHUNKS_PER_W * CHUNK
E_PAD = EPW * NW

def _sc_degree_body(dst_hbm, ones_hbm, zeros_hbm, out_hbm, dstv, onesv, hist_sh):
    cid = lax.axis_index("c")
    sid = lax.axis_index("s")
    wid = cid * 16 + sid
    row0 = sid * ROWS_PER_SUBCORE
    pltpu.sync_copy(ones_hbm, onesv)
    pltpu.sync_copy(zeros_hbm, hist_sh.at[pl.ds(row0, ROWS_PER_SUBCORE)])
    plsc.subcore_barrier()

    def body(j, carry):
        base = wid * EPW + j * CHUNK
        pltpu.sync_copy(dst_hbm.at[pl.ds(base, CHUNK)], dstv)
        pltpu.sync_copy(onesv, hist_sh.at[dstv], add=True)
        return carry

    lax.fori_loop(0, CHUNKS_PER_W, body, 0)
    plsc.subcore_barrier()
    pltpu.sync_copy(hist_sh.at[pl.ds(row0, ROWS_PER_SUBCORE)],
                    out_hbm.at[cid, pl.ds(row0, ROWS_PER_SUBCORE)])


def _sc_aggregate_body(y_hbm, src_hbm, dst_hbm, zeros_hbm, out_hbm,
                       srcv, dstv, rows, acc_sh, sem):
    cid = lax.axis_index("c")
    sid = lax.axis_index("s")
    wid = cid * 16 + sid
    row0 = sid * ROWS_PER_SUBCORE
    pltpu.sync_copy(zeros_hbm, acc_sh.at[pl.ds(row0, ROWS_PER_SUBCORE)])
    plsc.subcore_barrier()

    def body(j, carry):
        base = wid * EPW + j * CHUNK
        pltpu.sync_copy(src_hbm.at[pl.ds(base, CHUNK)], srcv)
        pltpu.sync_copy(dst_hbm.at[pl.ds(base, CHUNK)], dstv)
        pltpu.async_copy(y_hbm.at[srcv], rows, sem).wait()
        pltpu.sync_copy(rows, acc_sh.at[dstv], add=True)
        return carry

    lax.fori_loop(0, CHUNKS_PER_W, body, 0)
    plsc.subcore_barrier()
    pltpu.sync_copy(acc_sh.at[pl.ds(row0, ROWS_PER_SUBCORE)],
                    out_hbm.at[cid, pl.ds(row0, ROWS_PER_SUBCORE)])


@functools.lru_cache(maxsize=None)
def _sc_kernels():
    mesh = plsc.VectorSubcoreMesh(core_axis_name="c", subcore_axis_name="s")
    sc_degree = pl.kernel(
        _sc_degree_body,
        out_type=jax.ShapeDtypeStruct((2, N_PAD), jnp.float32),
        mesh=mesh,
        scratch_types=[
            pltpu.VMEM((CHUNK,), jnp.int32),
            pltpu.VMEM((CHUNK,), jnp.float32),
            pltpu.VMEM_SHARED((N_PAD,), jnp.float32),
        ],
    )
    sc_aggregate = pl.kernel(
        _sc_aggregate_body,
        out_type=jax.ShapeDtypeStruct((2, N_PAD, D), jnp.float32),
        mesh=mesh,
        scratch_types=[
            pltpu.VMEM((CHUNK,), jnp.int32),
            pltpu.VMEM((CHUNK,), jnp.int32),
            pltpu.VMEM((CHUNK, D), jnp.float32),
            pltpu.VMEM_SHARED((N_PAD, D), jnp.float32),
            pltpu.SemaphoreType.DMA,
        ],
    )
    return sc_degree, sc_aggregate


_BLK = 1024
_GRID = N_PAD // _BLK


def _dinv_of(hist_blk):
    deg = hist_blk[0, :] + hist_blk[1, :] + 1.0
    return lax.rsqrt(deg)


def _tc_first_body(hist_ref, x_ref, w_ref, y_ref):
    dinv = _dinv_of(hist_ref[...])
    h = jnp.dot(x_ref[...], w_ref[...], preferred_element_type=jnp.float32)
    y_ref[...] = h * dinv[:, None]


def _tc_mid_body(hist_ref, agg_ref, y_ref, b_ref, w_ref, o_ref):
    dinv = _dinv_of(hist_ref[...])
    s = agg_ref[0] + agg_ref[1] + y_ref[...]
    x1 = jnp.maximum(s * dinv[:, None] + b_ref[...], 0.0)
    o_ref[...] = jnp.dot(x1, w_ref[...], preferred_element_type=jnp.float32) * dinv[:, None]


def _tc_last_body(hist_ref, agg_ref, y_ref, b_ref, o_ref):
    dinv = _dinv_of(hist_ref[...])
    s = agg_ref[0] + agg_ref[1] + y_ref[...]
    o_ref[...] = s * dinv[:, None] + b_ref[...]


_hist_spec = pl.BlockSpec((2, _BLK), lambda i: (0, i))
_row_spec = pl.BlockSpec((_BLK, D), lambda i: (i, 0))
_agg_spec = pl.BlockSpec((2, _BLK, D), lambda i: (0, i, 0))
_full_spec = pl.BlockSpec((D, D), lambda i: (0, 0))
_bias_spec = pl.BlockSpec((1, D), lambda i: (0, 0))
_out_shape = jax.ShapeDtypeStruct((N_PAD, D), jnp.float32)

_tc_first = pl.pallas_call(
    _tc_first_body, grid=(_GRID,),
    in_specs=[_hist_spec, _row_spec, _full_spec],
    out_specs=_row_spec, out_shape=_out_shape)

_tc_mid = pl.pallas_call(
    _tc_mid_body, grid=(_GRID,),
    in_specs=[_hist_spec, _agg_spec, _row_spec, _bias_spec, _full_spec],
    out_specs=_row_spec, out_shape=_out_shape)

_tc_last = pl.pallas_call(
    _tc_last_body, grid=(_GRID,),
    in_specs=[_hist_spec, _agg_spec, _row_spec, _bias_spec],
    out_specs=_row_spec, out_shape=_out_shape)


def kernel(emb, edge_index, W1, b1, W2, b2):
    src = edge_index[0].astype(jnp.int32)
    dst = edge_index[1].astype(jnp.int32)

    n_extra = E_PAD - N_EDGES
    pad_src = (jnp.arange(n_extra, dtype=jnp.int32) * 97) % N_NODES
    pad_dst = N_NODES + (jnp.arange(n_extra, dtype=jnp.int32) % (N_PAD - N_NODES))
    src_p = jnp.concatenate([src, pad_src])
    dst_p = jnp.concatenate([dst, pad_dst])

    emb_p = jnp.pad(emb, ((0, N_PAD - N_NODES), (0, 0)))
    ones128 = jnp.ones((CHUNK,), jnp.float32)
    zrow = jnp.zeros((ROWS_PER_SUBCORE,), jnp.float32)
    zblk = jnp.zeros((ROWS_PER_SUBCORE, D), jnp.float32)
    b1r = b1.reshape(1, D)
    b2r = b2.reshape(1, D)

    sc_degree, sc_aggregate = _sc_kernels()
    hist = sc_degree(dst_p, ones128, zrow)
    y1 = _tc_first(hist, emb_p, W1)
    agg1 = sc_aggregate(y1, src_p, dst_p, zblk)
    y2 = _tc_mid(hist, agg1, y1, b1r, W2)
    agg2 = sc_aggregate(y2, src_p, dst_p, zblk)
    out = _tc_last(hist, agg2, y2, b2r)
    return out[:N_NODES]
